# Initial kernel scaffold; baseline (speedup 1.0000x reference)
import jax
import jax.numpy as jnp
from jax.experimental import pallas as pl


def kernel(logits):
    raise NotImplementedError("write your pallas kernel here")


# speedup vs baseline: 1.3163x; 1.3163x over previous
import jax
import jax.numpy as jnp
from jax.experimental import pallas as pl
from jax.experimental.pallas import tpu as pltpu

B0, B1, V = 32, 8, 128256
BIG = 2147483647


def _argmax_block(x_ref, o_ref):
    x = x_ref[0]
    m = jnp.max(x, axis=-1, keepdims=True)
    idx = jax.lax.broadcasted_iota(jnp.int32, x.shape, 1)
    cand = jnp.where(x == m, idx, jnp.int32(BIG))
    o_ref[0, 0, :] = jnp.min(cand, axis=-1)


def kernel(logits):
    out = pl.pallas_call(
        _argmax_block,
        grid=(B0,),
        in_specs=[pl.BlockSpec((1, B1, V), lambda i: (i, 0, 0))],
        out_specs=pl.BlockSpec((1, 1, B1), lambda i: (i, 0, 0)),
        out_shape=jax.ShapeDtypeStruct((B0, 1, B1), jnp.int32),
    )(logits)
    return out.reshape(B0, B1)
